# baseline (speedup 1.0000x reference)
import functools

import jax
import jax.numpy as jnp
from jax import lax
from jax.experimental import pallas as pl
from jax.experimental.pallas import tpu as pltpu
from jax.experimental.pallas import tpu_sc as plsc

N = 10000
E = 320000
D = 128
EDGE_D = 16
G = 64
L = 3
OUT = 128

NP = 10240
HALF = NP // 2
AGG_R = HALF
POISON = N
EP = 327680
EPS = EP // 16
C = 128
NCH = EPS // C
IBLK = 32

NB = 1024
NBLK = NP // NB


def _poison_pad_rows(y):
    g = pl.program_id(0)
    rowid = g * NB + lax.broadcasted_iota(jnp.int32, (NB, 1), 0)
    return jnp.where(rowid < N, y, -1e30)


def _enc_body(x_ref, w_ref, b_ref, o_ref):
    o_ref[...] = _poison_pad_rows(x_ref[...] @ w_ref[...] + b_ref[...])


def _edge_mm_body(a_ref, w_ref, b_ref, o_ref):
    o_ref[0] = a_ref[...] @ w_ref[0] + b_ref[0]


def _mlp_body(s_ref, h_ref, agg_ref, w1_ref, b1_ref, w2_ref, b2_ref, o_ref):
    z = h_ref[...] * s_ref[...] + agg_ref[...]
    z = jnp.maximum(z @ w1_ref[...] + b1_ref[...], 0.0)
    o_ref[...] = _poison_pad_rows(
        jnp.maximum(z @ w2_ref[...] + b2_ref[...], 0.0))


def _pool_body(bt_ref, h_ref, wfc_ref, bfc_ref, o_ref):
    b = pl.program_id(0)
    bid = bt_ref[0]
    ids = lax.broadcasted_iota(jnp.int32, (G, NB), 0)
    m = (ids == bid).astype(jnp.float32)
    part = jax.lax.dot(m, h_ref[...])

    @pl.when(b == 0)
    def _():
        o_ref[...] = part

    @pl.when(b > 0)
    def _():
        o_ref[...] = o_ref[...] + part

    @pl.when(b == NBLK - 1)
    def _():
        o_ref[...] = o_ref[...] @ wfc_ref[...] + bfc_ref[...]


def _make_edge_kernel(layer: int):
    mesh = plsc.VectorSubcoreMesh(
        core_axis_name="c", subcore_axis_name="s", num_cores=2, num_subcores=16
    )

    @functools.partial(
        pl.kernel,
        mesh=mesh,
        out_type=jax.ShapeDtypeStruct((NP, D), jnp.float32),
        scratch_types=[
            pltpu.VMEM((IBLK, 128), jnp.int32),
            pltpu.VMEM((IBLK, 128), jnp.int32),
            pltpu.VMEM((C, D), jnp.float32),
            pltpu.VMEM((C, D), jnp.float32),
            pltpu.VMEM((C, D), jnp.float32),
            pltpu.VMEM((C, D), jnp.float32),
            pltpu.VMEM((C, D), jnp.float32),
            pltpu.VMEM_SHARED((AGG_R, D), jnp.float32),
            pltpu.SemaphoreType.DMA,
            pltpu.SemaphoreType.DMA,
        ],
    )
    def edge_k(h_hbm, src_hbm, dst_hbm, e_hbm, out_hbm,
               srcblk, dstblk, rows0, rows1, ebuf0, ebuf1, msg, agg,
               sem0, sem1):
        c = lax.axis_index("c")
        s = lax.axis_index("s")
        rowbank = (rows0, rows1)
        ebank = (ebuf0, ebuf1)
        sems = (sem0, sem1)

        zv = jnp.zeros((16,), jnp.float32)

        def zero_body(r, _):
            for j in range(D // 16):
                ebuf0[r, pl.ds(j * 16, 16)] = zv
            return 0

        lax.fori_loop(0, C, zero_body, 0)
        zbase = s * (AGG_R // 16)
        pltpu.sync_copy(ebuf0, agg.at[pl.ds(zbase, C)])
        pltpu.sync_copy(ebuf0, agg.at[pl.ds(zbase + C, C)])
        pltpu.sync_copy(ebuf0.at[pl.ds(0, AGG_R // 16 - 2 * C)],
                        agg.at[pl.ds(zbase + 2 * C, AGG_R // 16 - 2 * C)])
        plsc.subcore_barrier()

        def stage_idx(t):
            rbase = pl.multiple_of(s * (EPS // 128) + t, 8)
            pltpu.sync_copy(src_hbm.at[c, pl.ds(rbase, IBLK)], srcblk)
            pltpu.sync_copy(dst_hbm.at[c, pl.ds(rbase, IBLK)], dstblk)

        def issue(t, p):
            j = t % IBLK
            ebase = s * EPS + t * C
            pltpu.async_copy(e_hbm.at[layer, pl.ds(ebase, C)],
                             ebank[p], sems[p])
            pltpu.async_copy(h_hbm.at[srcblk.at[j]], rowbank[p], sems[p])

        stage_idx(0)
        issue(0, 0)

        def pair_body(tt, _):
            for p in (0, 1):
                t = 2 * tt + p
                j = t % IBLK
                ebase = s * EPS + t * C
                pltpu.make_async_copy(e_hbm.at[layer, pl.ds(ebase, C)],
                                      ebank[p], sems[p]).wait()
                pltpu.make_async_copy(h_hbm.at[srcblk.at[j]],
                                      rowbank[p], sems[p]).wait()
                nxt_in_blk = (t + 1) % IBLK != 0

                @pl.when(nxt_in_blk)
                def _():
                    issue(t + 1, 1 - p)

                def row_body(rr, _):
                    for u in range(2):
                        r = 2 * rr + u
                        for g in range(D // 16):
                            sl = pl.ds(g * 16, 16)
                            msg[r, sl] = jnp.maximum(
                                rowbank[p][r, sl] + ebank[p][r, sl], 0.0)
                    return 0

                lax.fori_loop(0, C // 2, row_body, 0)
                pltpu.sync_copy(msg, agg.at[dstblk.at[j]], add=True)

                @pl.when(jnp.logical_and(jnp.logical_not(nxt_in_blk),
                                         t + 1 < NCH))
                def _():
                    stage_idx(t + 1)
                    issue(t + 1, 1 - p)

            return 0

        lax.fori_loop(0, NCH // 2, pair_body, 0)
        plsc.subcore_barrier()

        pltpu.sync_copy(agg.at[pl.ds(s * (HALF // 16), HALF // 16)],
                        out_hbm.at[pl.ds(c * HALF + s * (HALF // 16),
                                         HALF // 16)])

    return edge_k


def kernel(x, edge_index, edge_attr, batch, W_enc, b_enc, eps_all, W_edge_all,
           b_edge_all, W1_all, b1_all, W2_all, b2_all, W_fc, b_fc):
    f32 = jnp.float32
    x_pad = jnp.zeros((NP, D), f32).at[:N].set(x)
    order = jnp.argsort(edge_index[0])
    src = edge_index[0][order].astype(jnp.int32)
    dst = edge_index[1][order].astype(jnp.int32)
    edge_attr = edge_attr[order]
    src_p = jnp.concatenate([src, jnp.full((EP - E,), POISON, jnp.int32)])
    dst_p = jnp.concatenate([dst, jnp.full((EP - E,), NP, jnp.int32)])
    spread = jnp.arange(EP, dtype=jnp.int32) % HALF
    valid0 = dst_p < HALF
    valid1 = jnp.logical_and(dst_p >= HALF, dst_p < NP)
    src2 = jnp.stack([jnp.where(valid0, src_p, POISON),
                      jnp.where(valid1, src_p, POISON)]
                     ).reshape(2, EP // 128, 128)
    dst2 = jnp.stack([jnp.where(valid0, dst_p, spread),
                      jnp.where(valid1, dst_p - HALF, spread)]
                     ).reshape(2, EP // 128, 128)
    ea_pad = jnp.zeros((EP, EDGE_D), f32).at[:E].set(edge_attr)
    batch_pad = jnp.concatenate(
        [batch.astype(jnp.int32), jnp.full((NP - N,), G, jnp.int32)]
    ).reshape(NBLK, 1, NB)
    b_enc2 = b_enc.reshape(1, D)
    b_edge2 = b_edge_all.reshape(L, 1, D)
    b_fc2 = b_fc.reshape(1, OUT)

    h = pl.pallas_call(
        _enc_body,
        grid=(NBLK,),
        in_specs=[
            pl.BlockSpec((NB, D), lambda b: (b, 0)),
            pl.BlockSpec((D, D), lambda b: (0, 0)),
            pl.BlockSpec((1, D), lambda b: (0, 0)),
        ],
        out_specs=pl.BlockSpec((NB, D), lambda b: (b, 0)),
        out_shape=jax.ShapeDtypeStruct((NP, D), f32),
    )(x_pad, W_enc, b_enc2)

    EB = 2048
    e_all = pl.pallas_call(
        _edge_mm_body,
        grid=(L, EP // EB),
        in_specs=[
            pl.BlockSpec((EB, EDGE_D), lambda i, b: (b, 0)),
            pl.BlockSpec((1, EDGE_D, D), lambda i, b: (i, 0, 0)),
            pl.BlockSpec((1, 1, D), lambda i, b: (i, 0, 0)),
        ],
        out_specs=pl.BlockSpec((1, EB, D), lambda i, b: (i, b, 0)),
        out_shape=jax.ShapeDtypeStruct((L, EP, D), f32),
    )(ea_pad, W_edge_all, b_edge2)

    for i in range(L):
        agg = _make_edge_kernel(i)(h, src2, dst2, e_all)
        scale = jnp.full((1, D), 1.0, f32) + eps_all[i]
        h = pl.pallas_call(
            _mlp_body,
            grid=(NBLK,),
            in_specs=[
                pl.BlockSpec((1, D), lambda b: (0, 0)),
                pl.BlockSpec((NB, D), lambda b: (b, 0)),
                pl.BlockSpec((NB, D), lambda b: (b, 0)),
                pl.BlockSpec((D, D), lambda b: (0, 0)),
                pl.BlockSpec((1, D), lambda b: (0, 0)),
                pl.BlockSpec((D, D), lambda b: (0, 0)),
                pl.BlockSpec((1, D), lambda b: (0, 0)),
            ],
            out_specs=pl.BlockSpec((NB, D), lambda b: (b, 0)),
            out_shape=jax.ShapeDtypeStruct((NP, D), f32),
        )(scale, h, agg, W1_all[i], b1_all[i].reshape(1, D),
          W2_all[i], b2_all[i].reshape(1, D))

    out = pl.pallas_call(
        _pool_body,
        grid=(NBLK,),
        in_specs=[
            pl.BlockSpec((1, 1, NB), lambda b: (b, 0, 0)),
            pl.BlockSpec((NB, D), lambda b: (b, 0)),
            pl.BlockSpec((D, OUT), lambda b: (0, 0)),
            pl.BlockSpec((1, OUT), lambda b: (0, 0)),
        ],
        out_specs=pl.BlockSpec((G, OUT), lambda b: (0, 0)),
        out_shape=jax.ShapeDtypeStruct((G, OUT), f32),
    )(batch_pad, h, W_fc, b_fc2)
    return out


# speedup vs baseline: 1.0151x; 1.0151x over previous
import functools

import jax
import jax.numpy as jnp
from jax import lax
from jax.experimental import pallas as pl
from jax.experimental.pallas import tpu as pltpu
from jax.experimental.pallas import tpu_sc as plsc

N = 10000
E = 320000
D = 128
EDGE_D = 16
G = 64
L = 3
OUT = 128

NP = 10240
HALF = NP // 2
AGG_R = HALF
POISON = N
EP = 327680
EPS = EP // 16
C = 128
NCH = EPS // C
IBLK = 32

NB = 1024
NBLK = NP // NB


def _poison_pad_rows(y):
    g = pl.program_id(0)
    rowid = g * NB + lax.broadcasted_iota(jnp.int32, (NB, 1), 0)
    return jnp.where(rowid < N, y, -1e30)


def _enc_body(x_ref, w_ref, b_ref, o_ref):
    o_ref[...] = _poison_pad_rows(x_ref[...] @ w_ref[...] + b_ref[...])


def _edge_mm_body(a_ref, w_ref, b_ref, o_ref):
    o_ref[0] = a_ref[...] @ w_ref[0] + b_ref[0]


def _mlp_body(s_ref, h_ref, agg_ref, w1_ref, b1_ref, w2_ref, b2_ref, o_ref):
    z = h_ref[...] * s_ref[...] + agg_ref[...]
    z = jnp.maximum(z @ w1_ref[...] + b1_ref[...], 0.0)
    o_ref[...] = _poison_pad_rows(
        jnp.maximum(z @ w2_ref[...] + b2_ref[...], 0.0))


def _pool_body(bt_ref, h_ref, wfc_ref, bfc_ref, o_ref):
    b = pl.program_id(0)
    bid = bt_ref[0]
    ids = lax.broadcasted_iota(jnp.int32, (G, NB), 0)
    m = (ids == bid).astype(jnp.float32)
    part = jax.lax.dot(m, h_ref[...])

    @pl.when(b == 0)
    def _():
        o_ref[...] = part

    @pl.when(b > 0)
    def _():
        o_ref[...] = o_ref[...] + part

    @pl.when(b == NBLK - 1)
    def _():
        o_ref[...] = o_ref[...] @ wfc_ref[...] + bfc_ref[...]


def _make_edge_kernel(layer: int):
    mesh = plsc.VectorSubcoreMesh(
        core_axis_name="c", subcore_axis_name="s", num_cores=2, num_subcores=16
    )

    @functools.partial(
        pl.kernel,
        mesh=mesh,
        out_type=jax.ShapeDtypeStruct((NP, D), jnp.float32),
        scratch_types=[
            pltpu.VMEM((IBLK, 128), jnp.int32),
            pltpu.VMEM((IBLK, 128), jnp.int32),
            pltpu.VMEM((C, D), jnp.float32),
            pltpu.VMEM((C, D), jnp.float32),
            pltpu.VMEM((C, D), jnp.float32),
            pltpu.VMEM((C, D), jnp.float32),
            pltpu.VMEM((C, D), jnp.float32),
            pltpu.VMEM_SHARED((AGG_R, D), jnp.float32),
            pltpu.SemaphoreType.DMA,
            pltpu.SemaphoreType.DMA,
        ],
    )
    def edge_k(h_hbm, src_hbm, dst_hbm, e_hbm, out_hbm,
               srcblk, dstblk, rows0, rows1, ebuf0, ebuf1, msg, agg,
               sem0, sem1):
        c = lax.axis_index("c")
        s = lax.axis_index("s")
        rowbank = (rows0, rows1)
        ebank = (ebuf0, ebuf1)
        sems = (sem0, sem1)

        zv = jnp.zeros((16,), jnp.float32)

        def zero_body(r, _):
            for j in range(D // 16):
                ebuf0[r, pl.ds(j * 16, 16)] = zv
            return 0

        lax.fori_loop(0, C, zero_body, 0)
        zbase = s * (AGG_R // 16)
        pltpu.sync_copy(ebuf0, agg.at[pl.ds(zbase, C)])
        pltpu.sync_copy(ebuf0, agg.at[pl.ds(zbase + C, C)])
        pltpu.sync_copy(ebuf0.at[pl.ds(0, AGG_R // 16 - 2 * C)],
                        agg.at[pl.ds(zbase + 2 * C, AGG_R // 16 - 2 * C)])
        plsc.subcore_barrier()

        def stage_idx(t):
            rbase = pl.multiple_of(s * (EPS // 128) + t, 8)
            pltpu.sync_copy(src_hbm.at[c, pl.ds(rbase, IBLK)], srcblk)
            pltpu.sync_copy(dst_hbm.at[c, pl.ds(rbase, IBLK)], dstblk)

        def issue(t, p):
            j = t % IBLK
            ebase = s * EPS + t * C
            pltpu.async_copy(e_hbm.at[layer, pl.ds(ebase, C)],
                             ebank[p], sems[p])
            pltpu.async_copy(h_hbm.at[srcblk.at[j]], rowbank[p], sems[p])

        stage_idx(0)
        issue(0, 0)

        def pair_body(tt, _):
            for p in (0, 1):
                t = 2 * tt + p
                j = t % IBLK
                ebase = s * EPS + t * C
                pltpu.make_async_copy(e_hbm.at[layer, pl.ds(ebase, C)],
                                      ebank[p], sems[p]).wait()
                pltpu.make_async_copy(h_hbm.at[srcblk.at[j]],
                                      rowbank[p], sems[p]).wait()
                nxt_in_blk = (t + 1) % IBLK != 0

                @pl.when(nxt_in_blk)
                def _():
                    issue(t + 1, 1 - p)

                def row_body(rr, _):
                    for u in range(2):
                        r = 2 * rr + u
                        for g in range(D // 16):
                            sl = pl.ds(g * 16, 16)
                            msg[r, sl] = jnp.maximum(
                                rowbank[p][r, sl] + ebank[p][r, sl], 0.0)
                    return 0

                lax.fori_loop(0, C // 2, row_body, 0)
                pltpu.sync_copy(msg, agg.at[dstblk.at[j]], add=True)

                @pl.when(jnp.logical_and(jnp.logical_not(nxt_in_blk),
                                         t + 1 < NCH))
                def _():
                    stage_idx(t + 1)
                    issue(t + 1, 1 - p)

            return 0

        lax.fori_loop(0, NCH // 2, pair_body, 0)
        plsc.subcore_barrier()

        pltpu.sync_copy(agg.at[pl.ds(s * (HALF // 16), HALF // 16)],
                        out_hbm.at[pl.ds(c * HALF + s * (HALF // 16),
                                         HALF // 16)])

    return edge_k


def kernel(x, edge_index, edge_attr, batch, W_enc, b_enc, eps_all, W_edge_all,
           b_edge_all, W1_all, b1_all, W2_all, b2_all, W_fc, b_fc):
    f32 = jnp.float32
    x_pad = jnp.zeros((NP, D), f32).at[:N].set(x)
    src = edge_index[0].astype(jnp.int32)
    dst = edge_index[1].astype(jnp.int32)
    src_p = jnp.concatenate([src, jnp.full((EP - E,), POISON, jnp.int32)])
    dst_p = jnp.concatenate([dst, jnp.full((EP - E,), NP, jnp.int32)])
    spread = jnp.arange(EP, dtype=jnp.int32) % HALF
    valid0 = dst_p < HALF
    valid1 = jnp.logical_and(dst_p >= HALF, dst_p < NP)
    src2 = jnp.stack([jnp.where(valid0, src_p, POISON),
                      jnp.where(valid1, src_p, POISON)]
                     ).reshape(2, EP // 128, 128)
    dst2 = jnp.stack([jnp.where(valid0, dst_p, spread),
                      jnp.where(valid1, dst_p - HALF, spread)]
                     ).reshape(2, EP // 128, 128)
    ea_pad = jnp.zeros((EP, EDGE_D), f32).at[:E].set(edge_attr)
    batch_pad = jnp.concatenate(
        [batch.astype(jnp.int32), jnp.full((NP - N,), G, jnp.int32)]
    ).reshape(NBLK, 1, NB)
    b_enc2 = b_enc.reshape(1, D)
    b_edge2 = b_edge_all.reshape(L, 1, D)
    b_fc2 = b_fc.reshape(1, OUT)

    h = pl.pallas_call(
        _enc_body,
        grid=(NBLK,),
        in_specs=[
            pl.BlockSpec((NB, D), lambda b: (b, 0)),
            pl.BlockSpec((D, D), lambda b: (0, 0)),
            pl.BlockSpec((1, D), lambda b: (0, 0)),
        ],
        out_specs=pl.BlockSpec((NB, D), lambda b: (b, 0)),
        out_shape=jax.ShapeDtypeStruct((NP, D), f32),
    )(x_pad, W_enc, b_enc2)

    EB = 2048
    e_all = pl.pallas_call(
        _edge_mm_body,
        grid=(L, EP // EB),
        in_specs=[
            pl.BlockSpec((EB, EDGE_D), lambda i, b: (b, 0)),
            pl.BlockSpec((1, EDGE_D, D), lambda i, b: (i, 0, 0)),
            pl.BlockSpec((1, 1, D), lambda i, b: (i, 0, 0)),
        ],
        out_specs=pl.BlockSpec((1, EB, D), lambda i, b: (i, b, 0)),
        out_shape=jax.ShapeDtypeStruct((L, EP, D), f32),
    )(ea_pad, W_edge_all, b_edge2)

    for i in range(L):
        agg = _make_edge_kernel(i)(h, src2, dst2, e_all)
        scale = jnp.full((1, D), 1.0, f32) + eps_all[i]
        h = pl.pallas_call(
            _mlp_body,
            grid=(NBLK,),
            in_specs=[
                pl.BlockSpec((1, D), lambda b: (0, 0)),
                pl.BlockSpec((NB, D), lambda b: (b, 0)),
                pl.BlockSpec((NB, D), lambda b: (b, 0)),
                pl.BlockSpec((D, D), lambda b: (0, 0)),
                pl.BlockSpec((1, D), lambda b: (0, 0)),
                pl.BlockSpec((D, D), lambda b: (0, 0)),
                pl.BlockSpec((1, D), lambda b: (0, 0)),
            ],
            out_specs=pl.BlockSpec((NB, D), lambda b: (b, 0)),
            out_shape=jax.ShapeDtypeStruct((NP, D), f32),
        )(scale, h, agg, W1_all[i], b1_all[i].reshape(1, D),
          W2_all[i], b2_all[i].reshape(1, D))

    out = pl.pallas_call(
        _pool_body,
        grid=(NBLK,),
        in_specs=[
            pl.BlockSpec((1, 1, NB), lambda b: (b, 0, 0)),
            pl.BlockSpec((NB, D), lambda b: (b, 0)),
            pl.BlockSpec((D, OUT), lambda b: (0, 0)),
            pl.BlockSpec((1, OUT), lambda b: (0, 0)),
        ],
        out_specs=pl.BlockSpec((G, OUT), lambda b: (0, 0)),
        out_shape=jax.ShapeDtypeStruct((G, OUT), f32),
    )(batch_pad, h, W_fc, b_fc2)
    return out


# speedup vs baseline: 21.7801x; 21.4559x over previous
import functools

import jax
import jax.numpy as jnp
from jax import lax
from jax.experimental import pallas as pl
from jax.experimental.pallas import tpu as pltpu
from jax.experimental.pallas import tpu_sc as plsc

N = 10000
E = 320000
D = 128
EDGE_D = 16
G = 64
L = 3
OUT = 128

NP = 10240
HALF = NP // 2
AGG_R = HALF
POISON = N
EP = 327680
EPS = EP // 16
C = 128
NCH = EPS // C
IBLK = 32

NB = 1024
NBLK = NP // NB


def _poison_pad_rows(y):
    g = pl.program_id(0)
    rowid = g * NB + lax.broadcasted_iota(jnp.int32, (NB, 1), 0)
    return jnp.where(rowid < N, y, -1e30)


def _enc_body(x_ref, w_ref, b_ref, o_ref):
    o_ref[...] = _poison_pad_rows(x_ref[...] @ w_ref[...] + b_ref[...])


def _edge_mm_body(a_ref, w_ref, b_ref, o_ref):
    o_ref[0] = a_ref[...] @ w_ref[0] + b_ref[0]


def _mlp_body(s_ref, h_ref, agg_ref, w1_ref, b1_ref, w2_ref, b2_ref, o_ref):
    z = h_ref[...] * s_ref[...] + agg_ref[...]
    z = jnp.maximum(z @ w1_ref[...] + b1_ref[...], 0.0)
    o_ref[...] = _poison_pad_rows(
        jnp.maximum(z @ w2_ref[...] + b2_ref[...], 0.0))


def _pool_body(bt_ref, h_ref, wfc_ref, bfc_ref, o_ref):
    b = pl.program_id(0)
    bid = bt_ref[0]
    ids = lax.broadcasted_iota(jnp.int32, (G, NB), 0)
    m = (ids == bid).astype(jnp.float32)
    part = jax.lax.dot(m, h_ref[...])

    @pl.when(b == 0)
    def _():
        o_ref[...] = part

    @pl.when(b > 0)
    def _():
        o_ref[...] = o_ref[...] + part

    @pl.when(b == NBLK - 1)
    def _():
        o_ref[...] = o_ref[...] @ wfc_ref[...] + bfc_ref[...]


def _make_edge_kernel(layer: int):
    mesh = plsc.VectorSubcoreMesh(
        core_axis_name="c", subcore_axis_name="s", num_cores=2, num_subcores=16
    )

    @functools.partial(
        pl.kernel,
        mesh=mesh,
        out_type=jax.ShapeDtypeStruct((NP, D), jnp.float32),
        scratch_types=[
            pltpu.VMEM((IBLK, 128), jnp.int32),
            pltpu.VMEM((IBLK, 128), jnp.int32),
            pltpu.VMEM((C, D), jnp.float32),
            pltpu.VMEM((C, D), jnp.float32),
            pltpu.VMEM((C, D), jnp.float32),
            pltpu.VMEM((C, D), jnp.float32),
            pltpu.VMEM((C, D), jnp.float32),
            pltpu.VMEM_SHARED((AGG_R, D), jnp.float32),
            pltpu.SemaphoreType.DMA,
            pltpu.SemaphoreType.DMA,
        ],
    )
    def edge_k(h_hbm, src_hbm, dst_hbm, e_hbm, out_hbm,
               srcblk, dstblk, rows0, rows1, ebuf0, ebuf1, msg, agg,
               sem0, sem1):
        c = lax.axis_index("c")
        s = lax.axis_index("s")
        rowbank = (rows0, rows1)
        ebank = (ebuf0, ebuf1)
        sems = (sem0, sem1)

        zv = jnp.zeros((16,), jnp.float32)

        def zero_body(r, _):
            for j in range(D // 16):
                ebuf0[r, pl.ds(j * 16, 16)] = zv
            return 0

        lax.fori_loop(0, C, zero_body, 0)
        zbase = s * (AGG_R // 16)
        pltpu.sync_copy(ebuf0, agg.at[pl.ds(zbase, C)])
        pltpu.sync_copy(ebuf0, agg.at[pl.ds(zbase + C, C)])
        pltpu.sync_copy(ebuf0.at[pl.ds(0, AGG_R // 16 - 2 * C)],
                        agg.at[pl.ds(zbase + 2 * C, AGG_R // 16 - 2 * C)])
        plsc.subcore_barrier()

        def stage_idx(t):
            rbase = pl.multiple_of(s * (EPS // 128) + t, 8)
            pltpu.sync_copy(src_hbm.at[c, pl.ds(rbase, IBLK)], srcblk)
            pltpu.sync_copy(dst_hbm.at[c, pl.ds(rbase, IBLK)], dstblk)

        def issue(t, p):
            j = t % IBLK
            ebase = s * EPS + t * C
            pltpu.async_copy(e_hbm.at[layer, pl.ds(ebase, C)],
                             ebank[p], sems[p])
            pltpu.async_copy(h_hbm.at[srcblk.at[j]], rowbank[p], sems[p])

        stage_idx(0)
        issue(0, 0)

        def pair_body(tt, _):
            for p in (0, 1):
                t = 2 * tt + p
                j = t % IBLK
                ebase = s * EPS + t * C
                pltpu.make_async_copy(e_hbm.at[layer, pl.ds(ebase, C)],
                                      ebank[p], sems[p]).wait()
                pltpu.make_async_copy(h_hbm.at[srcblk.at[j]],
                                      rowbank[p], sems[p]).wait()
                nxt_in_blk = (t + 1) % IBLK != 0

                @pl.when(nxt_in_blk)
                def _():
                    issue(t + 1, 1 - p)

                def row_body(rr, _):
                    for u in range(2):
                        r = 2 * rr + u
                        for g in range(D // 16):
                            sl = pl.ds(g * 16, 16)
                            msg[r, sl] = jnp.maximum(
                                rowbank[p][r, sl] + ebank[p][r, sl], 0.0)
                    return 0

                lax.fori_loop(0, C // 2, row_body, 0)
                pltpu.sync_copy(msg, agg.at[dstblk.at[j]], add=True)

                @pl.when(jnp.logical_and(jnp.logical_not(nxt_in_blk),
                                         t + 1 < NCH))
                def _():
                    stage_idx(t + 1)
                    issue(t + 1, 1 - p)

            return 0

        lax.fori_loop(0, NCH // 2, pair_body, 0)
        plsc.subcore_barrier()

        pltpu.sync_copy(agg.at[pl.ds(s * (HALF // 16), HALF // 16)],
                        out_hbm.at[pl.ds(c * HALF + s * (HALF // 16),
                                         HALF // 16)])

    return edge_k


def kernel(x, edge_index, edge_attr, batch, W_enc, b_enc, eps_all, W_edge_all,
           b_edge_all, W1_all, b1_all, W2_all, b2_all, W_fc, b_fc):
    f32 = jnp.float32
    x_pad = jnp.zeros((NP, D), f32).at[:N].set(x)
    src = edge_index[0].astype(jnp.int32)
    dst = edge_index[1].astype(jnp.int32)
    src_p = jnp.concatenate([src, jnp.full((EP - E,), POISON, jnp.int32)])
    dst_p = jnp.concatenate([dst, jnp.full((EP - E,), NP, jnp.int32)])
    spread = jnp.arange(EP, dtype=jnp.int32) % HALF
    poison_spread = POISON + (jnp.arange(EP, dtype=jnp.int32) % (NP - N))
    valid0 = dst_p < HALF
    valid1 = jnp.logical_and(dst_p >= HALF, dst_p < NP)
    src2 = jnp.stack([jnp.where(valid0, src_p, poison_spread),
                      jnp.where(valid1, src_p, poison_spread)]
                     ).reshape(2, EP // 128, 128)
    dst2 = jnp.stack([jnp.where(valid0, dst_p, spread),
                      jnp.where(valid1, dst_p - HALF, spread)]
                     ).reshape(2, EP // 128, 128)
    ea_pad = jnp.zeros((EP, EDGE_D), f32).at[:E].set(edge_attr)
    batch_pad = jnp.concatenate(
        [batch.astype(jnp.int32), jnp.full((NP - N,), G, jnp.int32)]
    ).reshape(NBLK, 1, NB)
    b_enc2 = b_enc.reshape(1, D)
    b_edge2 = b_edge_all.reshape(L, 1, D)
    b_fc2 = b_fc.reshape(1, OUT)

    h = pl.pallas_call(
        _enc_body,
        grid=(NBLK,),
        in_specs=[
            pl.BlockSpec((NB, D), lambda b: (b, 0)),
            pl.BlockSpec((D, D), lambda b: (0, 0)),
            pl.BlockSpec((1, D), lambda b: (0, 0)),
        ],
        out_specs=pl.BlockSpec((NB, D), lambda b: (b, 0)),
        out_shape=jax.ShapeDtypeStruct((NP, D), f32),
    )(x_pad, W_enc, b_enc2)

    EB = 2048
    e_all = pl.pallas_call(
        _edge_mm_body,
        grid=(L, EP // EB),
        in_specs=[
            pl.BlockSpec((EB, EDGE_D), lambda i, b: (b, 0)),
            pl.BlockSpec((1, EDGE_D, D), lambda i, b: (i, 0, 0)),
            pl.BlockSpec((1, 1, D), lambda i, b: (i, 0, 0)),
        ],
        out_specs=pl.BlockSpec((1, EB, D), lambda i, b: (i, b, 0)),
        out_shape=jax.ShapeDtypeStruct((L, EP, D), f32),
    )(ea_pad, W_edge_all, b_edge2)

    for i in range(L):
        agg = _make_edge_kernel(i)(h, src2, dst2, e_all)
        scale = jnp.full((1, D), 1.0, f32) + eps_all[i]
        h = pl.pallas_call(
            _mlp_body,
            grid=(NBLK,),
            in_specs=[
                pl.BlockSpec((1, D), lambda b: (0, 0)),
                pl.BlockSpec((NB, D), lambda b: (b, 0)),
                pl.BlockSpec((NB, D), lambda b: (b, 0)),
                pl.BlockSpec((D, D), lambda b: (0, 0)),
                pl.BlockSpec((1, D), lambda b: (0, 0)),
                pl.BlockSpec((D, D), lambda b: (0, 0)),
                pl.BlockSpec((1, D), lambda b: (0, 0)),
            ],
            out_specs=pl.BlockSpec((NB, D), lambda b: (b, 0)),
            out_shape=jax.ShapeDtypeStruct((NP, D), f32),
        )(scale, h, agg, W1_all[i], b1_all[i].reshape(1, D),
          W2_all[i], b2_all[i].reshape(1, D))

    out = pl.pallas_call(
        _pool_body,
        grid=(NBLK,),
        in_specs=[
            pl.BlockSpec((1, 1, NB), lambda b: (b, 0, 0)),
            pl.BlockSpec((NB, D), lambda b: (b, 0)),
            pl.BlockSpec((D, OUT), lambda b: (0, 0)),
            pl.BlockSpec((1, OUT), lambda b: (0, 0)),
        ],
        out_specs=pl.BlockSpec((G, OUT), lambda b: (0, 0)),
        out_shape=jax.ShapeDtypeStruct((G, OUT), f32),
    )(batch_pad, h, W_fc, b_fc2)
    return out


# speedup vs baseline: 23.8417x; 1.0947x over previous
import functools

import jax
import jax.numpy as jnp
from jax import lax
from jax.experimental import pallas as pl
from jax.experimental.pallas import tpu as pltpu
from jax.experimental.pallas import tpu_sc as plsc

N = 10000
E = 320000
D = 128
EDGE_D = 16
G = 64
L = 3
OUT = 128

NP = 10240
HALF = NP // 2
AGG_R = HALF
POISON = N
EP = 327680
EPS = EP // 16
C = 128
NCH = EPS // C
IBLK = 32

NB = 1024
NBLK = NP // NB


def _poison_pad_rows(y):
    g = pl.program_id(0)
    rowid = g * NB + lax.broadcasted_iota(jnp.int32, (NB, 1), 0)
    return jnp.where(rowid < N, y, -1e30)


def _enc_body(x_ref, w_ref, b_ref, o_ref):
    o_ref[...] = _poison_pad_rows(x_ref[...] @ w_ref[...] + b_ref[...])


def _edge_mm_body(a_ref, w_ref, b_ref, o_ref):
    o_ref[0] = a_ref[...] @ w_ref[0] + b_ref[0]


def _mlp_body(s_ref, h_ref, agg_ref, w1_ref, b1_ref, w2_ref, b2_ref, o_ref):
    z = h_ref[...] * s_ref[...] + agg_ref[...]
    z = jnp.maximum(z @ w1_ref[...] + b1_ref[...], 0.0)
    o_ref[...] = _poison_pad_rows(
        jnp.maximum(z @ w2_ref[...] + b2_ref[...], 0.0))


def _pool_body(bt_ref, h_ref, wfc_ref, bfc_ref, o_ref):
    b = pl.program_id(0)
    bid = bt_ref[0]
    ids = lax.broadcasted_iota(jnp.int32, (G, NB), 0)
    m = (ids == bid).astype(jnp.float32)
    part = jax.lax.dot(m, h_ref[...])

    @pl.when(b == 0)
    def _():
        o_ref[...] = part

    @pl.when(b > 0)
    def _():
        o_ref[...] = o_ref[...] + part

    @pl.when(b == NBLK - 1)
    def _():
        o_ref[...] = o_ref[...] @ wfc_ref[...] + bfc_ref[...]


def _make_edge_kernel(layer: int):
    mesh = plsc.VectorSubcoreMesh(
        core_axis_name="c", subcore_axis_name="s", num_cores=2, num_subcores=16
    )

    @functools.partial(
        pl.kernel,
        mesh=mesh,
        out_type=jax.ShapeDtypeStruct((NP, D), jnp.float32),
        scratch_types=[
            pltpu.VMEM((IBLK, 128), jnp.int32),
            pltpu.VMEM((IBLK, 128), jnp.int32),
            pltpu.VMEM((C, D), jnp.float32),
            pltpu.VMEM((C, D), jnp.float32),
            pltpu.VMEM((C, D), jnp.float32),
            pltpu.VMEM((C, D), jnp.float32),
            pltpu.VMEM((C, D), jnp.float32),
            pltpu.VMEM_SHARED((AGG_R, D), jnp.float32),
            pltpu.SemaphoreType.DMA,
            pltpu.SemaphoreType.DMA,
        ],
    )
    def edge_k(h_hbm, src_hbm, dst_hbm, e_hbm, out_hbm,
               srcblk, dstblk, rows0, rows1, ebuf0, ebuf1, msg, agg,
               sem0, sem1):
        c = lax.axis_index("c")
        s = lax.axis_index("s")
        rowbank = (rows0, rows1)
        ebank = (ebuf0, ebuf1)
        sems = (sem0, sem1)

        zv = jnp.zeros((16,), jnp.float32)

        def zero_body(r, _):
            for j in range(D // 16):
                ebuf0[r, pl.ds(j * 16, 16)] = zv
            return 0

        lax.fori_loop(0, C, zero_body, 0)
        zbase = s * (AGG_R // 16)
        pltpu.sync_copy(ebuf0, agg.at[pl.ds(zbase, C)])
        pltpu.sync_copy(ebuf0, agg.at[pl.ds(zbase + C, C)])
        pltpu.sync_copy(ebuf0.at[pl.ds(0, AGG_R // 16 - 2 * C)],
                        agg.at[pl.ds(zbase + 2 * C, AGG_R // 16 - 2 * C)])
        plsc.subcore_barrier()

        def stage_idx(t):
            rbase = pl.multiple_of(s * (EPS // 128) + t, 8)
            pltpu.sync_copy(src_hbm.at[c, pl.ds(rbase, IBLK)], srcblk)
            pltpu.sync_copy(dst_hbm.at[c, pl.ds(rbase, IBLK)], dstblk)

        def issue(t, p):
            j = t % IBLK
            ebase = s * EPS + t * C
            pltpu.async_copy(e_hbm.at[layer, pl.ds(ebase, C)],
                             ebank[p], sems[p])
            pltpu.async_copy(h_hbm.at[srcblk.at[j]], rowbank[p], sems[p])

        stage_idx(0)
        issue(0, 0)

        def pair_body(tt, _):
            for p in (0, 1):
                t = 2 * tt + p
                j = t % IBLK
                ebase = s * EPS + t * C
                pltpu.make_async_copy(e_hbm.at[layer, pl.ds(ebase, C)],
                                      ebank[p], sems[p]).wait()
                pltpu.make_async_copy(h_hbm.at[srcblk.at[j]],
                                      rowbank[p], sems[p]).wait()
                nxt_in_blk = (t + 1) % IBLK != 0

                @pl.when(nxt_in_blk)
                def _():
                    issue(t + 1, 1 - p)

                def row_body(rr, _):
                    for u in range(2):
                        r = 2 * rr + u
                        for g in range(D // 16):
                            sl = pl.ds(g * 16, 16)
                            msg[r, sl] = jnp.maximum(
                                rowbank[p][r, sl] + ebank[p][r, sl], 0.0)
                    return 0

                lax.fori_loop(0, C // 2, row_body, 0)
                pltpu.sync_copy(msg, agg.at[dstblk.at[j]], add=True)

                @pl.when(jnp.logical_and(jnp.logical_not(nxt_in_blk),
                                         t + 1 < NCH))
                def _():
                    stage_idx(t + 1)
                    issue(t + 1, 1 - p)

            return 0

        lax.fori_loop(0, NCH // 2, pair_body, 0)
        plsc.subcore_barrier()

        pltpu.sync_copy(agg.at[pl.ds(s * (HALF // 16), HALF // 16)],
                        out_hbm.at[pl.ds(c * HALF + s * (HALF // 16),
                                         HALF // 16)])

    return edge_k


def kernel(x, edge_index, edge_attr, batch, W_enc, b_enc, eps_all, W_edge_all,
           b_edge_all, W1_all, b1_all, W2_all, b2_all, W_fc, b_fc):
    f32 = jnp.float32
    x_pad = jnp.zeros((NP, D), f32).at[:N].set(x)
    src = edge_index[0].astype(jnp.int32)
    dst = edge_index[1].astype(jnp.int32)
    src_p = jnp.concatenate([src, jnp.full((EP - E,), POISON, jnp.int32)])
    dst_p = jnp.concatenate([dst, jnp.full((EP - E,), NP, jnp.int32)])
    spread = jnp.arange(EP, dtype=jnp.int32) % HALF
    poison_spread = POISON + (jnp.arange(EP, dtype=jnp.int32) % (NP - N))
    valid0 = dst_p < HALF
    valid1 = jnp.logical_and(dst_p >= HALF, dst_p < NP)
    src2 = jnp.stack([jnp.where(valid0, src_p, poison_spread),
                      jnp.where(valid1, src_p, poison_spread)]
                     ).reshape(2, EP // 128, 128)
    dst2 = jnp.stack([jnp.where(valid0, dst_p, spread),
                      jnp.where(valid1, dst_p - HALF, spread)]
                     ).reshape(2, EP // 128, 128)
    ea_pad = jnp.zeros((EP, EDGE_D), f32).at[:E].set(edge_attr)
    batch_pad = jnp.concatenate(
        [batch.astype(jnp.int32), jnp.full((NP - N,), G, jnp.int32)]
    ).reshape(NBLK, 1, NB)
    b_enc2 = b_enc.reshape(1, D)
    b_edge2 = b_edge_all.reshape(L, 1, D)
    b_fc2 = b_fc.reshape(1, OUT)

    h = pl.pallas_call(
        _enc_body,
        grid=(NBLK,),
        in_specs=[
            pl.BlockSpec((NB, D), lambda b: (b, 0)),
            pl.BlockSpec((D, D), lambda b: (0, 0)),
            pl.BlockSpec((1, D), lambda b: (0, 0)),
        ],
        out_specs=pl.BlockSpec((NB, D), lambda b: (b, 0)),
        out_shape=jax.ShapeDtypeStruct((NP, D), f32),
    )(x_pad, W_enc, b_enc2)

    EB = 2048

    def edge_feat(i):
        return pl.pallas_call(
            _edge_mm_body,
            grid=(EP // EB,),
            in_specs=[
                pl.BlockSpec((EB, EDGE_D), lambda b: (b, 0)),
                pl.BlockSpec((1, EDGE_D, D), lambda b: (0, 0, 0)),
                pl.BlockSpec((1, 1, D), lambda b: (0, 0, 0)),
            ],
            out_specs=pl.BlockSpec((1, EB, D), lambda b: (0, b, 0)),
            out_shape=jax.ShapeDtypeStruct((1, EP, D), f32),
        )(ea_pad, W_edge_all[i:i + 1], b_edge2[i:i + 1])

    e_cur = edge_feat(0)

    for i in range(L):
        agg = _make_edge_kernel(0)(h, src2, dst2, e_cur)
        if i + 1 < L:
            e_cur = edge_feat(i + 1)
        scale = jnp.full((1, D), 1.0, f32) + eps_all[i]
        h = pl.pallas_call(
            _mlp_body,
            grid=(NBLK,),
            in_specs=[
                pl.BlockSpec((1, D), lambda b: (0, 0)),
                pl.BlockSpec((NB, D), lambda b: (b, 0)),
                pl.BlockSpec((NB, D), lambda b: (b, 0)),
                pl.BlockSpec((D, D), lambda b: (0, 0)),
                pl.BlockSpec((1, D), lambda b: (0, 0)),
                pl.BlockSpec((D, D), lambda b: (0, 0)),
                pl.BlockSpec((1, D), lambda b: (0, 0)),
            ],
            out_specs=pl.BlockSpec((NB, D), lambda b: (b, 0)),
            out_shape=jax.ShapeDtypeStruct((NP, D), f32),
        )(scale, h, agg, W1_all[i], b1_all[i].reshape(1, D),
          W2_all[i], b2_all[i].reshape(1, D))

    out = pl.pallas_call(
        _pool_body,
        grid=(NBLK,),
        in_specs=[
            pl.BlockSpec((1, 1, NB), lambda b: (b, 0, 0)),
            pl.BlockSpec((NB, D), lambda b: (b, 0)),
            pl.BlockSpec((D, OUT), lambda b: (0, 0)),
            pl.BlockSpec((1, OUT), lambda b: (0, 0)),
        ],
        out_specs=pl.BlockSpec((G, OUT), lambda b: (0, 0)),
        out_shape=jax.ShapeDtypeStruct((G, OUT), f32),
    )(batch_pad, h, W_fc, b_fc2)
    return out


# speedup vs baseline: 24.0594x; 1.0091x over previous
import functools

import jax
import jax.numpy as jnp
from jax import lax
from jax.experimental import pallas as pl
from jax.experimental.pallas import tpu as pltpu
from jax.experimental.pallas import tpu_sc as plsc

N = 10000
E = 320000
D = 128
EDGE_D = 16
G = 64
L = 3
OUT = 128

NP = 10240
HALF = NP // 2
AGG_R = HALF
POISON = N
EP = 327680
EPS = EP // 16
C = 128
NCH = EPS // C
IBLK = 80

NB = 1024
NBLK = NP // NB


def _poison_pad_rows(y):
    g = pl.program_id(0)
    rowid = g * NB + lax.broadcasted_iota(jnp.int32, (NB, 1), 0)
    return jnp.where(rowid < N, y, -1e30)


def _enc_body(x_ref, w_ref, b_ref, o_ref):
    o_ref[...] = _poison_pad_rows(x_ref[...] @ w_ref[...] + b_ref[...])


def _edge_mm_body(a_ref, w_ref, b_ref, o_ref):
    o_ref[0] = a_ref[...] @ w_ref[0] + b_ref[0]


def _mlp_body(s_ref, h_ref, agg_ref, w1_ref, b1_ref, w2_ref, b2_ref, o_ref):
    z = h_ref[...] * s_ref[...] + agg_ref[...]
    z = jnp.maximum(z @ w1_ref[...] + b1_ref[...], 0.0)
    o_ref[...] = _poison_pad_rows(
        jnp.maximum(z @ w2_ref[...] + b2_ref[...], 0.0))


def _pool_body(bt_ref, h_ref, wfc_ref, bfc_ref, o_ref):
    b = pl.program_id(0)
    bid = bt_ref[0]
    ids = lax.broadcasted_iota(jnp.int32, (G, NB), 0)
    m = (ids == bid).astype(jnp.float32)
    part = jax.lax.dot(m, h_ref[...])

    @pl.when(b == 0)
    def _():
        o_ref[...] = part

    @pl.when(b > 0)
    def _():
        o_ref[...] = o_ref[...] + part

    @pl.when(b == NBLK - 1)
    def _():
        o_ref[...] = o_ref[...] @ wfc_ref[...] + bfc_ref[...]


def _make_edge_kernel(layer: int):
    mesh = plsc.VectorSubcoreMesh(
        core_axis_name="c", subcore_axis_name="s", num_cores=2, num_subcores=16
    )

    @functools.partial(
        pl.kernel,
        mesh=mesh,
        out_type=jax.ShapeDtypeStruct((NP, D), jnp.float32),
        scratch_types=[
            pltpu.VMEM((IBLK, 128), jnp.int32),
            pltpu.VMEM((IBLK, 128), jnp.int32),
            pltpu.VMEM((C, D), jnp.float32),
            pltpu.VMEM((C, D), jnp.float32),
            pltpu.VMEM((C, D), jnp.float32),
            pltpu.VMEM((C, D), jnp.float32),
            pltpu.VMEM_SHARED((AGG_R, D), jnp.float32),
            pltpu.SemaphoreType.DMA,
            pltpu.SemaphoreType.DMA,
        ],
    )
    def edge_k(h_hbm, src_hbm, dst_hbm, e_hbm, out_hbm,
               srcblk, dstblk, rows0, rows1, ebuf0, ebuf1, agg,
               sem0, sem1):
        c = lax.axis_index("c")
        s = lax.axis_index("s")
        rowbank = (rows0, rows1)
        ebank = (ebuf0, ebuf1)
        sems = (sem0, sem1)

        zv = jnp.zeros((16,), jnp.float32)

        def zero_body(r, _):
            for j in range(D // 16):
                ebuf0[r, pl.ds(j * 16, 16)] = zv
            return 0

        lax.fori_loop(0, C, zero_body, 0)
        zbase = s * (AGG_R // 16)
        pltpu.sync_copy(ebuf0, agg.at[pl.ds(zbase, C)])
        pltpu.sync_copy(ebuf0, agg.at[pl.ds(zbase + C, C)])
        pltpu.sync_copy(ebuf0.at[pl.ds(0, AGG_R // 16 - 2 * C)],
                        agg.at[pl.ds(zbase + 2 * C, AGG_R // 16 - 2 * C)])
        plsc.subcore_barrier()

        def stage_idx(t):
            rbase = pl.multiple_of(s * (EPS // 128) + t, 8)
            pltpu.sync_copy(src_hbm.at[c, pl.ds(rbase, IBLK)], srcblk)
            pltpu.sync_copy(dst_hbm.at[c, pl.ds(rbase, IBLK)], dstblk)

        def issue(t, p):
            j = t % IBLK
            ebase = s * EPS + t * C
            pltpu.async_copy(e_hbm.at[layer, pl.ds(ebase, C)],
                             ebank[p], sems[p])
            pltpu.async_copy(h_hbm.at[srcblk.at[j]], rowbank[p], sems[p])

        stage_idx(0)
        issue(0, 0)

        def pair_body(tt, _):
            for p in (0, 1):
                t = 2 * tt + p
                j = t % IBLK
                ebase = s * EPS + t * C
                pltpu.make_async_copy(e_hbm.at[layer, pl.ds(ebase, C)],
                                      ebank[p], sems[p]).wait()
                pltpu.make_async_copy(h_hbm.at[srcblk.at[j]],
                                      rowbank[p], sems[p]).wait()
                nxt_in_blk = (t + 1) % IBLK != 0

                @pl.when(nxt_in_blk)
                def _():
                    issue(t + 1, 1 - p)

                def row_body(rr, _):
                    for u in range(2):
                        r = 2 * rr + u
                        for g in range(D // 16):
                            sl = pl.ds(g * 16, 16)
                            rowbank[p][r, sl] = jnp.maximum(
                                rowbank[p][r, sl] + ebank[p][r, sl], 0.0)
                    return 0

                lax.fori_loop(0, C // 2, row_body, 0)
                pltpu.sync_copy(rowbank[p], agg.at[dstblk.at[j]], add=True)

                @pl.when(jnp.logical_and(jnp.logical_not(nxt_in_blk),
                                         t + 1 < NCH))
                def _():
                    stage_idx(t + 1)
                    issue(t + 1, 1 - p)

            return 0

        lax.fori_loop(0, NCH // 2, pair_body, 0)
        plsc.subcore_barrier()

        pltpu.sync_copy(agg.at[pl.ds(s * (HALF // 16), HALF // 16)],
                        out_hbm.at[pl.ds(c * HALF + s * (HALF // 16),
                                         HALF // 16)])

    return edge_k


def kernel(x, edge_index, edge_attr, batch, W_enc, b_enc, eps_all, W_edge_all,
           b_edge_all, W1_all, b1_all, W2_all, b2_all, W_fc, b_fc):
    f32 = jnp.float32
    x_pad = jnp.zeros((NP, D), f32).at[:N].set(x)
    src = edge_index[0].astype(jnp.int32)
    dst = edge_index[1].astype(jnp.int32)
    src_p = jnp.concatenate([src, jnp.full((EP - E,), POISON, jnp.int32)])
    dst_p = jnp.concatenate([dst, jnp.full((EP - E,), NP, jnp.int32)])
    spread = jnp.arange(EP, dtype=jnp.int32) % HALF
    poison_spread = POISON + (jnp.arange(EP, dtype=jnp.int32) % (NP - N))
    valid0 = dst_p < HALF
    valid1 = jnp.logical_and(dst_p >= HALF, dst_p < NP)
    src2 = jnp.stack([jnp.where(valid0, src_p, poison_spread),
                      jnp.where(valid1, src_p, poison_spread)]
                     ).reshape(2, EP // 128, 128)
    dst2 = jnp.stack([jnp.where(valid0, dst_p, spread),
                      jnp.where(valid1, dst_p - HALF, spread)]
                     ).reshape(2, EP // 128, 128)
    ea_pad = jnp.zeros((EP, EDGE_D), f32).at[:E].set(edge_attr)
    batch_pad = jnp.concatenate(
        [batch.astype(jnp.int32), jnp.full((NP - N,), G, jnp.int32)]
    ).reshape(NBLK, 1, NB)
    b_enc2 = b_enc.reshape(1, D)
    b_edge2 = b_edge_all.reshape(L, 1, D)
    b_fc2 = b_fc.reshape(1, OUT)

    h = pl.pallas_call(
        _enc_body,
        grid=(NBLK,),
        in_specs=[
            pl.BlockSpec((NB, D), lambda b: (b, 0)),
            pl.BlockSpec((D, D), lambda b: (0, 0)),
            pl.BlockSpec((1, D), lambda b: (0, 0)),
        ],
        out_specs=pl.BlockSpec((NB, D), lambda b: (b, 0)),
        out_shape=jax.ShapeDtypeStruct((NP, D), f32),
    )(x_pad, W_enc, b_enc2)

    EB = 2048

    def edge_feat(i):
        return pl.pallas_call(
            _edge_mm_body,
            grid=(EP // EB,),
            in_specs=[
                pl.BlockSpec((EB, EDGE_D), lambda b: (b, 0)),
                pl.BlockSpec((1, EDGE_D, D), lambda b: (0, 0, 0)),
                pl.BlockSpec((1, 1, D), lambda b: (0, 0, 0)),
            ],
            out_specs=pl.BlockSpec((1, EB, D), lambda b: (0, b, 0)),
            out_shape=jax.ShapeDtypeStruct((1, EP, D), f32),
        )(ea_pad, W_edge_all[i:i + 1], b_edge2[i:i + 1])

    e_cur = edge_feat(0)

    for i in range(L):
        agg = _make_edge_kernel(0)(h, src2, dst2, e_cur)
        if i + 1 < L:
            e_cur = edge_feat(i + 1)
        scale = jnp.full((1, D), 1.0, f32) + eps_all[i]
        h = pl.pallas_call(
            _mlp_body,
            grid=(NBLK,),
            in_specs=[
                pl.BlockSpec((1, D), lambda b: (0, 0)),
                pl.BlockSpec((NB, D), lambda b: (b, 0)),
                pl.BlockSpec((NB, D), lambda b: (b, 0)),
                pl.BlockSpec((D, D), lambda b: (0, 0)),
                pl.BlockSpec((1, D), lambda b: (0, 0)),
                pl.BlockSpec((D, D), lambda b: (0, 0)),
                pl.BlockSpec((1, D), lambda b: (0, 0)),
            ],
            out_specs=pl.BlockSpec((NB, D), lambda b: (b, 0)),
            out_shape=jax.ShapeDtypeStruct((NP, D), f32),
        )(scale, h, agg, W1_all[i], b1_all[i].reshape(1, D),
          W2_all[i], b2_all[i].reshape(1, D))

    out = pl.pallas_call(
        _pool_body,
        grid=(NBLK,),
        in_specs=[
            pl.BlockSpec((1, 1, NB), lambda b: (b, 0, 0)),
            pl.BlockSpec((NB, D), lambda b: (b, 0)),
            pl.BlockSpec((D, OUT), lambda b: (0, 0)),
            pl.BlockSpec((1, OUT), lambda b: (0, 0)),
        ],
        out_specs=pl.BlockSpec((G, OUT), lambda b: (0, 0)),
        out_shape=jax.ShapeDtypeStruct((G, OUT), f32),
    )(batch_pad, h, W_fc, b_fc2)
    return out
